# Initial kernel scaffold; baseline (speedup 1.0000x reference)
import jax
import jax.numpy as jnp
from jax.experimental import pallas as pl


def kernel(x, Wg, W1, b1, W2, b2):
    raise NotImplementedError("write your pallas kernel here")


# speedup vs baseline: 4.3738x; 4.3738x over previous
import functools

import jax
import jax.numpy as jnp
from jax import lax
from jax.experimental import pallas as pl
from jax.experimental.pallas import tpu as pltpu
from jax.experimental.pallas import tpu_sc as plsc

E = 8
K = 2
CAP_RATIO = 1.05

NC = 2
NS = 16
NW = NC * NS
L = 16

_MESH = plsc.VectorSubcoreMesh(core_axis_name="c", subcore_axis_name="s")


def _route_call(xf, wgt, T, D, cap, capP):
  R = E * capP
  LN = 128

  def body(x_ref, wg_ref, dsc0_ref, dsc1_ref, dg0_ref, dg1_ref,
           w0_ref, w1_ref, oh_s, tb_s):
    xx = x_ref[...]
    logits = jnp.dot(xx, wg_ref[...], preferred_element_type=jnp.float32)
    lane = lax.broadcasted_iota(jnp.int32, (T, LN), 1)
    valid = lane < E
    logits = jnp.where(valid, logits, jnp.float32(-1e30))
    m = jnp.max(logits, axis=1, keepdims=True)
    ex = jnp.where(valid, jnp.exp(logits - m), 0.0)
    gates = ex / jnp.sum(ex, axis=1, keepdims=True)

    g0 = jnp.max(gates, axis=1, keepdims=True)
    e0 = jnp.min(jnp.where((gates == g0) & valid, lane, LN), axis=1,
                 keepdims=True)
    gates1 = jnp.where(lane == e0, -1.0, gates)
    g1 = jnp.max(gates1, axis=1, keepdims=True)
    e1 = jnp.min(jnp.where((gates1 == g1) & valid, lane, LN), axis=1,
                 keepdims=True)

    ti = lax.broadcasted_iota(jnp.int32, (LN, LN), 0)
    tj = lax.broadcasted_iota(jnp.int32, (LN, LN), 1)
    tri = (tj < ti).astype(jnp.float32)

    def occ_count(e_idx, prior):
      oh_s[...] = (lane == e_idx).astype(jnp.float32)

      def chunk(c, carry):
        blk = oh_s[pl.ds(c * LN, LN), :]
        occ = carry + jnp.dot(tri, blk, preferred_element_type=jnp.float32)
        tb_s[pl.ds(c * LN, LN), :] = jnp.sum(occ * blk, axis=1, keepdims=True)
        return carry + jnp.sum(blk, axis=0, keepdims=True)

      lax.fori_loop(0, T // LN, chunk, prior)
      return tb_s[...]

    capf = jnp.float32(cap)

    tb0 = occ_count(e0, jnp.zeros((1, LN), jnp.float32))
    adm0 = tb0 < capf
    p0 = tb0.astype(jnp.int32)
    slot0 = e0 * capP + p0
    dsc0_ref[...] = jnp.where(adm0, slot0, R)
    dg0_ref[...] = jnp.where(adm0, slot0, e0 * capP + (capP - 1))
    w0_ref[...] = jnp.where(adm0, g0, 0.0)
    prior = jnp.sum(oh_s[...] * jnp.where(adm0, 1.0, 0.0), axis=0,
                    keepdims=True)

    tb1 = occ_count(e1, prior)
    adm1 = tb1 < capf
    p1 = tb1.astype(jnp.int32)
    slot1 = e1 * capP + p1
    dsc1_ref[...] = jnp.where(adm1, slot1, R)
    dg1_ref[...] = jnp.where(adm1, slot1, e1 * capP + (capP - 1))
    w1_ref[...] = jnp.where(adm1, g1, 0.0)

  i32 = jax.ShapeDtypeStruct((T, 1), jnp.int32)
  f32 = jax.ShapeDtypeStruct((T, 1), jnp.float32)
  return pl.pallas_call(
      body,
      out_shape=(i32, i32, i32, i32, f32, f32),
      scratch_shapes=[
          pltpu.VMEM((T, LN), jnp.float32),
          pltpu.VMEM((T, 1), jnp.float32),
      ],
  )(xf, wgt)


def _dispatch_call(dsc0, dsc1, xf, T, D, capP):
  R = E * capP
  RP = R + L
  rows_per_tile = R // NW

  @functools.partial(
      pl.kernel,
      out_type=jax.ShapeDtypeStruct((R, D), jnp.float32),
      mesh=_MESH,
      scratch_types=[
          pltpu.VMEM((RP,), jnp.int32),
          pltpu.VMEM((T,), jnp.int32),
          pltpu.VMEM((T,), jnp.int32),
          pltpu.VMEM((rows_per_tile, D), jnp.float32),
          pltpu.SemaphoreType.DMA,
      ],
      compiler_params=pltpu.CompilerParams(needs_layout_passes=False),
  )
  def k(d0_hbm, d1_hbm, x_hbm, xd_hbm, slots_v, d0_v, d1_v, rows_v, sem):
    wid = lax.axis_index("s") * NC + lax.axis_index("c")
    pltpu.sync_copy(d0_hbm, d0_v)
    pltpu.sync_copy(d1_hbm, d1_v)

    def zero(i, c):
      slots_v[pl.ds(i * L, L)] = jnp.zeros((L,), jnp.int32)
      return c

    lax.fori_loop(0, RP // L, zero, 0)

    def scat(i, c):
      tok = i * L + lax.iota(jnp.int32, L)
      plsc.store_scatter(slots_v, [d0_v[pl.ds(i * L, L)]], tok)
      plsc.store_scatter(slots_v, [d1_v[pl.ds(i * L, L)]], tok)
      return c

    lax.fori_loop(0, T // L, scat, 0)

    base = wid * rows_per_tile
    idx = slots_v.at[pl.ds(base, rows_per_tile)]
    pltpu.async_copy(x_hbm.at[idx], rows_v, sem).wait()
    pltpu.sync_copy(rows_v, xd_hbm.at[pl.ds(base, rows_per_tile)])

  return k(dsc0, dsc1, xf)


def _experts_call(xd, W1, b1, W2, b2, capP, D, H):
  HT = 1536
  nh = H // HT

  def body(xd_ref, w1_ref, b1_ref, w2_ref, b2_ref, out_ref):
    hid = pl.program_id(1)
    x = xd_ref[0]
    h = lax.dot_general(x, w1_ref[0], (((1,), (1,)), ((), ())),
                        preferred_element_type=jnp.float32)
    h = h + b1_ref[0]
    h = 0.5 * h * (1.0 + lax.erf(h * 0.7071067811865476))
    part = lax.dot_general(h, w2_ref[0], (((1,), (1,)), ((), ())),
                           preferred_element_type=jnp.float32)

    @pl.when(hid == 0)
    def _():
      out_ref[0] = jnp.broadcast_to(b2_ref[0], (capP, D))

    out_ref[0] += part

  return pl.pallas_call(
      body,
      grid=(E, nh),
      in_specs=[
          pl.BlockSpec((1, capP, D), lambda e, h: (e, 0, 0)),
          pl.BlockSpec((1, HT, D), lambda e, h: (e, h, 0)),
          pl.BlockSpec((1, 1, HT), lambda e, h: (e, 0, h)),
          pl.BlockSpec((1, D, HT), lambda e, h: (e, 0, h)),
          pl.BlockSpec((1, 1, D), lambda e, h: (e, 0, 0)),
      ],
      out_specs=pl.BlockSpec((1, capP, D), lambda e, h: (e, 0, 0)),
      out_shape=jax.ShapeDtypeStruct((E, capP, D), jnp.float32),
      compiler_params=pltpu.CompilerParams(
          dimension_semantics=("parallel", "arbitrary")),
  )(xd, W1, b1.reshape(E, 1, H), W2, b2.reshape(E, 1, D))


def _combine_call(dg0, dg1, w0, w1, yd, T, D):
  tok_per_tile = T // NW

  @functools.partial(
      pl.kernel,
      out_type=jax.ShapeDtypeStruct((T, D), jnp.float32),
      mesh=_MESH,
      scratch_types=[
          pltpu.VMEM((tok_per_tile,), jnp.int32),
          pltpu.VMEM((tok_per_tile,), jnp.int32),
          pltpu.VMEM((tok_per_tile, D), jnp.float32),
          pltpu.VMEM((tok_per_tile, D), jnp.float32),
          pltpu.VMEM((tok_per_tile,), jnp.float32),
          pltpu.VMEM((tok_per_tile,), jnp.float32),
          pltpu.SemaphoreType.DMA,
      ],
      compiler_params=pltpu.CompilerParams(needs_layout_passes=False),
  )
  def k(dg0_hbm, dg1_hbm, w0_hbm, w1_hbm, yd_hbm, out_hbm,
        i0_v, i1_v, r0_v, r1_v, w0_v, w1_v, sem):
    wid = lax.axis_index("s") * NC + lax.axis_index("c")
    base = wid * tok_per_tile
    pltpu.sync_copy(dg0_hbm.at[pl.ds(base, tok_per_tile)], i0_v)
    pltpu.sync_copy(dg1_hbm.at[pl.ds(base, tok_per_tile)], i1_v)
    pltpu.sync_copy(w0_hbm.at[pl.ds(base, tok_per_tile)], w0_v)
    pltpu.sync_copy(w1_hbm.at[pl.ds(base, tok_per_tile)], w1_v)
    pltpu.async_copy(yd_hbm.at[i0_v], r0_v, sem).wait()
    pltpu.async_copy(yd_hbm.at[i1_v], r1_v, sem).wait()

    def tok(j, c):
      jv = jnp.full((L,), j, jnp.int32)
      a = plsc.load_gather(w0_v, [jv])
      b = plsc.load_gather(w1_v, [jv])
      for v in range(D // L):
        sl = pl.ds(v * L, L)
        r0_v[j, sl] = a * r0_v[j, sl] + b * r1_v[j, sl]
      return c

    lax.fori_loop(0, tok_per_tile, tok, 0)
    pltpu.sync_copy(r0_v, out_hbm.at[pl.ds(base, tok_per_tile)])

  return k(dg0, dg1, w0, w1, yd)


def kernel(x, Wg, W1, b1, W2, b2):
  B, S, D = x.shape
  H = W1.shape[1]
  T = B * S
  cap = round(K * T * CAP_RATIO / E)
  capP = ((cap + 1 + 7) // 8) * 8
  R = E * capP
  assert T % (NW * 8) == 0 and R % (NW * 8) == 0 and D % L == 0

  xf = x.reshape(T, D)
  wgt = jnp.zeros((D, 128), jnp.float32).at[:, :E].set(Wg.T)

  dsc0, dsc1, dg0, dg1, w0, w1 = _route_call(xf, wgt, T, D, cap, capP)

  xd = _dispatch_call(dsc0.reshape(T), dsc1.reshape(T), xf, T, D, capP)

  yd = _experts_call(xd.reshape(E, capP, D), W1, b1, W2, b2, capP, D, H)

  out = _combine_call(dg0.reshape(T), dg1.reshape(T),
                      w0.reshape(T), w1.reshape(T),
                      yd.reshape(R, D), T, D)
  return out.reshape(B, S, D)
